# Initial kernel scaffold; baseline (speedup 1.0000x reference)
import jax
import jax.numpy as jnp
from jax.experimental import pallas as pl


def kernel(center_words, pos_context_words, neg_context_words, W_in, W_out):
    raise NotImplementedError("write your pallas kernel here")


# speedup vs baseline: 4.6331x; 4.6331x over previous
import functools

import jax
import jax.numpy as jnp
from jax import lax
from jax.experimental import pallas as pl
from jax.experimental.pallas import tpu as pltpu
from jax.experimental.pallas import tpu_sc as plsc

VOCAB = 1000000
DIM = 64
B = 16384
K = 20

NC = 2
NS = 16
NW = NC * NS
BPW = B // NW
CB = 32
NCHUNK = BPW // CB
NEG_STREAM = 128
NEG_STREAMS = CB * K // NEG_STREAM


def _partial64(a_ref, arow, b_ref, brow):
    acc = a_ref[arow, pl.ds(0, 16)] * b_ref[brow, pl.ds(0, 16)]
    for j in range(1, 4):
        acc = acc + a_ref[arow, pl.ds(j * 16, 16)] * b_ref[brow, pl.ds(j * 16, 16)]
    return acc


def _rowsum16(tr):
    acc = tr[pl.ds(0, 16)]
    for l in range(1, 16):
        acc = acc + tr[pl.ds(l * 16, 16)]
    return acc


def _sg_body(cw, pw, nw, w_in, w_out, pos_out, neg_out,
             ci, pi, ni, vin, vout, vneg, po, no, tr, sem):
    wid = lax.axis_index("s") * NC + lax.axis_index("c")

    def chunk_body(c, _):
        base = wid * BPW + c * CB
        pltpu.sync_copy(cw.at[pl.ds(base, CB)], ci)
        pltpu.sync_copy(pw.at[pl.ds(base, CB)], pi)
        pltpu.sync_copy(nw.at[pl.ds(base * K, CB * K)], ni)
        pltpu.async_copy(w_in.at[ci], vin, sem).wait()
        pltpu.async_copy(w_out.at[pi], vout, sem).wait()
        for s in range(NEG_STREAMS):
            pltpu.async_copy(
                w_out.at[ni.at[pl.ds(s * NEG_STREAM, NEG_STREAM)]],
                vneg.at[pl.ds(s * NEG_STREAM, NEG_STREAM)],
                sem,
            ).wait()

        lanes = lax.iota(jnp.int32, 16)

        def pos_group(g, _):
            def dot_body(l, _):
                r = g * 16 + l
                p = _partial64(vin, r, vout, r)
                plsc.store_scatter(tr, [lanes * 16 + l], p)
                return 0

            lax.fori_loop(0, 16, dot_body, 0)
            po[pl.ds(g * 16, 16)] = _rowsum16(tr)
            return 0

        lax.fori_loop(0, CB // 16, pos_group, 0)

        def neg_group(g, _):
            def dot_body(l, _):
                r = g * 16 + l
                p = _partial64(vin, r // K, vneg, r)
                plsc.store_scatter(tr, [lanes * 16 + l], p)
                return 0

            lax.fori_loop(0, 16, dot_body, 0)
            no[pl.ds(g * 16, 16)] = _rowsum16(tr)
            return 0

        lax.fori_loop(0, CB * K // 16, neg_group, 0)
        pltpu.sync_copy(po, pos_out.at[pl.ds(base, CB)])
        pltpu.sync_copy(no, neg_out.at[pl.ds(base * K, CB * K)])
        return 0

    lax.fori_loop(0, NCHUNK, chunk_body, 0)


_sg_call = functools.partial(
    pl.kernel,
    out_type=[
        jax.ShapeDtypeStruct((B,), jnp.float32),
        jax.ShapeDtypeStruct((B * K,), jnp.float32),
    ],
    mesh=plsc.VectorSubcoreMesh(core_axis_name="c", subcore_axis_name="s"),
    compiler_params=pltpu.CompilerParams(
        needs_layout_passes=False, use_tc_tiling_on_sc=False
    ),
    scratch_types=[
        pltpu.VMEM((CB,), jnp.int32),
        pltpu.VMEM((CB,), jnp.int32),
        pltpu.VMEM((CB * K,), jnp.int32),
        pltpu.VMEM((CB, DIM), jnp.float32),
        pltpu.VMEM((CB, DIM), jnp.float32),
        pltpu.VMEM((CB * K, DIM), jnp.float32),
        pltpu.VMEM((CB,), jnp.float32),
        pltpu.VMEM((CB * K,), jnp.float32),
        pltpu.VMEM((256,), jnp.float32),
        pltpu.SemaphoreType.DMA,
    ],
)(_sg_body)


def kernel(center_words, pos_context_words, neg_context_words, W_in, W_out):
    cw = center_words.astype(jnp.int32)
    pw = pos_context_words.astype(jnp.int32)
    nw = neg_context_words.astype(jnp.int32).reshape(B * K)
    pos_scores, neg_flat = _sg_call(cw, pw, nw, W_in, W_out)
    return pos_scores, neg_flat.reshape(B, K)


# speedup vs baseline: 4.8167x; 1.0396x over previous
import functools

import jax
import jax.numpy as jnp
from jax import lax
from jax.experimental import pallas as pl
from jax.experimental.pallas import tpu as pltpu
from jax.experimental.pallas import tpu_sc as plsc

VOCAB = 1000000
DIM = 64
B = 16384
K = 20

NC = 2
NS = 16
NW = NC * NS
BPW = B // NW
CB = 32
NCHUNK = BPW // CB


def _partial64(a_ref, arow, b_ref, brow):
    acc = a_ref[arow, pl.ds(0, 16)] * b_ref[brow, pl.ds(0, 16)]
    for j in range(1, 4):
        acc = acc + a_ref[arow, pl.ds(j * 16, 16)] * b_ref[brow, pl.ds(j * 16, 16)]
    return acc


def _partial64_3d(a_ref, b_ref, r):
    i = r // K
    k = r - i * K
    acc = a_ref[i, pl.ds(0, 16)] * b_ref[i, k, pl.ds(0, 16)]
    for j in range(1, 4):
        acc = acc + a_ref[i, pl.ds(j * 16, 16)] * b_ref[i, k, pl.ds(j * 16, 16)]
    return acc


def _rowsum16(tr):
    acc = tr[pl.ds(0, 16)]
    for l in range(1, 16):
        acc = acc + tr[pl.ds(l * 16, 16)]
    return acc


def _sg_body(cw, pw, nw, w_in, w_out, pos_out, neg_out,
             ci, pi, ni, vin, vout, vneg, po, no, tr, sem):
    wid = lax.axis_index("s") * NC + lax.axis_index("c")

    def chunk_body(c, _):
        base = wid * BPW + c * CB
        pltpu.sync_copy(cw.at[pl.ds(base, CB)], ci)
        pltpu.sync_copy(pw.at[pl.ds(base, CB)], pi)
        pltpu.sync_copy(nw.at[pl.ds(base, CB)], ni)
        copies = [
            pltpu.async_copy(w_in.at[ci], vin, sem),
            pltpu.async_copy(w_out.at[pi], vout, sem),
        ]
        for i in range(CB):
            copies.append(
                pltpu.async_copy(w_out.at[ni.at[i]], vneg.at[i], sem)
            )
        for cp in copies:
            cp.wait()

        lanes = lax.iota(jnp.int32, 16)

        def pos_group(g, _):
            def dot_body(l, _):
                r = g * 16 + l
                p = _partial64(vin, r, vout, r)
                plsc.store_scatter(tr, [lanes * 16 + l], p)
                return 0

            lax.fori_loop(0, 16, dot_body, 0)
            po[pl.ds(g * 16, 16)] = _rowsum16(tr)
            return 0

        lax.fori_loop(0, CB // 16, pos_group, 0)

        def neg_group(g, _):
            def dot_body(l, _):
                r = g * 16 + l
                p = _partial64_3d(vin, vneg, r)
                plsc.store_scatter(tr, [lanes * 16 + l], p)
                return 0

            lax.fori_loop(0, 16, dot_body, 0)
            no[pl.ds(g * 16, 16)] = _rowsum16(tr)
            return 0

        lax.fori_loop(0, CB * K // 16, neg_group, 0)
        pltpu.sync_copy(po, pos_out.at[pl.ds(base, CB)])
        pltpu.sync_copy(no, neg_out.at[pl.ds(base * K, CB * K)])
        return 0

    lax.fori_loop(0, NCHUNK, chunk_body, 0)


_sg_call = functools.partial(
    pl.kernel,
    out_type=[
        jax.ShapeDtypeStruct((B,), jnp.float32),
        jax.ShapeDtypeStruct((B * K,), jnp.float32),
    ],
    mesh=plsc.VectorSubcoreMesh(core_axis_name="c", subcore_axis_name="s"),
    compiler_params=pltpu.CompilerParams(
        needs_layout_passes=False, use_tc_tiling_on_sc=False
    ),
    scratch_types=[
        pltpu.VMEM((CB,), jnp.int32),
        pltpu.VMEM((CB,), jnp.int32),
        pltpu.VMEM((CB, K), jnp.int32),
        pltpu.VMEM((CB, DIM), jnp.float32),
        pltpu.VMEM((CB, DIM), jnp.float32),
        pltpu.VMEM((CB, K, DIM), jnp.float32),
        pltpu.VMEM((CB,), jnp.float32),
        pltpu.VMEM((CB * K,), jnp.float32),
        pltpu.VMEM((256,), jnp.float32),
        pltpu.SemaphoreType.DMA,
    ],
)(_sg_body)


def kernel(center_words, pos_context_words, neg_context_words, W_in, W_out):
    cw = center_words.astype(jnp.int32)
    pw = pos_context_words.astype(jnp.int32)
    nw = neg_context_words.astype(jnp.int32)
    pos_scores, neg_flat = _sg_call(cw, pw, nw, W_in, W_out)
    return pos_scores, neg_flat.reshape(B, K)


# speedup vs baseline: 5.0161x; 1.0414x over previous
import functools

import jax
import jax.numpy as jnp
from jax import lax
from jax.experimental import pallas as pl
from jax.experimental.pallas import tpu as pltpu
from jax.experimental.pallas import tpu_sc as plsc

VOCAB = 1000000
DIM = 64
B = 16384
K = 20

NC = 2
NS = 16
NW = NC * NS
BPW = B // NW
CB = 32
NCHUNK = BPW // CB


def _partial64(a_ref, arow, b_ref, brow):
    acc = a_ref[arow, pl.ds(0, 16)] * b_ref[brow, pl.ds(0, 16)]
    for j in range(1, 4):
        acc = acc + a_ref[arow, pl.ds(j * 16, 16)] * b_ref[brow, pl.ds(j * 16, 16)]
    return acc


def _rowsum16(tr):
    acc = tr[pl.ds(0, 16)]
    for l in range(1, 16):
        acc = acc + tr[pl.ds(l * 16, 16)]
    return acc


def _sg_body(cw, pw, nw, w_in, w_out, pos_out, neg_out,
             ci, pi, ni, vin2, vout2, vneg2, po, no, tr, sems):
    wid = lax.axis_index("s") * NC + lax.axis_index("c")
    wbase = wid * BPW

    pltpu.sync_copy(cw.at[pl.ds(wbase, BPW)], ci)
    pltpu.sync_copy(pw.at[pl.ds(wbase, BPW)], pi)
    pltpu.sync_copy(nw.at[pl.ds(wbase, BPW)], ni)

    def issue(c, p):
        sem = sems.at[p]
        pltpu.async_copy(w_in.at[ci.at[pl.ds(c * CB, CB)]], vin2.at[p], sem)
        pltpu.async_copy(w_out.at[pi.at[pl.ds(c * CB, CB)]], vout2.at[p], sem)

        def gi(i, _):
            pltpu.async_copy(w_out.at[ni.at[c * CB + i]], vneg2.at[p, i], sem)
            return 0

        lax.fori_loop(0, CB, gi, 0)

    def drain(c, p):
        sem = sems.at[p]
        pltpu.make_async_copy(w_in.at[ci.at[pl.ds(c * CB, CB)]], vin2.at[p], sem).wait()
        pltpu.make_async_copy(w_out.at[pi.at[pl.ds(c * CB, CB)]], vout2.at[p], sem).wait()

        def gw(i, _):
            pltpu.make_async_copy(w_out.at[ni.at[c * CB + i]], vneg2.at[p, i], sem).wait()
            return 0

        lax.fori_loop(0, CB, gw, 0)

    def compute(p):
        lanes = lax.iota(jnp.int32, 16)

        def pos_group(g, _):
            def dot_body(l, _):
                r = g * 16 + l
                plsc.store_scatter(tr, [lanes * 16 + l],
                                   _partial64(vin2.at[p], r, vout2.at[p], r))
                return 0

            lax.fori_loop(0, 16, dot_body, 0)
            po[pl.ds(g * 16, 16)] = _rowsum16(tr)
            return 0

        lax.fori_loop(0, CB // 16, pos_group, 0)

        def neg_group(g, _):
            def dot_body(l, _):
                r = g * 16 + l
                i = r // K
                k = r - i * K
                a = vin2.at[p]
                b = vneg2.at[p]
                acc = a[i, pl.ds(0, 16)] * b[i, k, pl.ds(0, 16)]
                for j in range(1, 4):
                    acc = acc + a[i, pl.ds(j * 16, 16)] * b[i, k, pl.ds(j * 16, 16)]
                plsc.store_scatter(tr, [lanes * 16 + l], acc)
                return 0

            lax.fori_loop(0, 16, dot_body, 0)
            no[pl.ds(g * 16, 16)] = _rowsum16(tr)
            return 0

        lax.fori_loop(0, CB * K // 16, neg_group, 0)

    issue(0, 0)

    def chunk_body(c, _):
        p = lax.rem(c, 2)

        @pl.when(c + 1 < NCHUNK)
        def _():
            issue(c + 1, lax.rem(c + 1, 2))

        drain(c, p)
        compute(p)
        base = wbase + c * CB
        pltpu.sync_copy(po, pos_out.at[pl.ds(base, CB)])
        pltpu.sync_copy(no, neg_out.at[pl.ds(base * K, CB * K)])
        return 0

    lax.fori_loop(0, NCHUNK, chunk_body, 0)


_sg_call = functools.partial(
    pl.kernel,
    out_type=[
        jax.ShapeDtypeStruct((B,), jnp.float32),
        jax.ShapeDtypeStruct((B * K,), jnp.float32),
    ],
    mesh=plsc.VectorSubcoreMesh(core_axis_name="c", subcore_axis_name="s"),
    compiler_params=pltpu.CompilerParams(
        needs_layout_passes=False, use_tc_tiling_on_sc=False
    ),
    scratch_types=[
        pltpu.VMEM((BPW,), jnp.int32),
        pltpu.VMEM((BPW,), jnp.int32),
        pltpu.VMEM((BPW, K), jnp.int32),
        pltpu.VMEM((2, CB, DIM), jnp.float32),
        pltpu.VMEM((2, CB, DIM), jnp.float32),
        pltpu.VMEM((2, CB, K, DIM), jnp.float32),
        pltpu.VMEM((CB,), jnp.float32),
        pltpu.VMEM((CB * K,), jnp.float32),
        pltpu.VMEM((256,), jnp.float32),
        pltpu.SemaphoreType.DMA((2,)),
    ],
)(_sg_body)


def kernel(center_words, pos_context_words, neg_context_words, W_in, W_out):
    cw = center_words.astype(jnp.int32)
    pw = pos_context_words.astype(jnp.int32)
    nw = neg_context_words.astype(jnp.int32)
    pos_scores, neg_flat = _sg_call(cw, pw, nw, W_in, W_out)
    return pos_scores, neg_flat.reshape(B, K)


# speedup vs baseline: 5.0401x; 1.0048x over previous
import functools

import jax
import jax.numpy as jnp
from jax import lax
from jax.experimental import pallas as pl
from jax.experimental.pallas import tpu as pltpu
from jax.experimental.pallas import tpu_sc as plsc

VOCAB = 1000000
DIM = 64
B = 16384
K = 20

NC = 2
NS = 16
NW = NC * NS
BPW = B // NW
CB = 32
NCHUNK = BPW // CB


def _partial64(a_ref, arow, b_ref, brow):
    acc = a_ref[arow, pl.ds(0, 16)] * b_ref[brow, pl.ds(0, 16)]
    for j in range(1, 4):
        acc = acc + a_ref[arow, pl.ds(j * 16, 16)] * b_ref[brow, pl.ds(j * 16, 16)]
    return acc


def _rowsum16(tr):
    acc = tr[pl.ds(0, 16)]
    for l in range(1, 16):
        acc = acc + tr[pl.ds(l * 16, 16)]
    return acc


def _sg_body(cw, pw, nw, w_in, w_out, pos_out, neg_out,
             ci, pi, ni, vin2, vout2, vneg2, po, no, tr, sems):
    wid = lax.axis_index("s") * NC + lax.axis_index("c")
    wbase = wid * BPW

    pltpu.sync_copy(cw.at[pl.ds(wbase, BPW)], ci)
    pltpu.sync_copy(pw.at[pl.ds(wbase, BPW)], pi)
    pltpu.sync_copy(nw.at[pl.ds(wbase, BPW)], ni)

    def issue(c, p):
        sem = sems.at[p]
        pltpu.async_copy(w_in.at[ci.at[pl.ds(c * CB, CB)]], vin2.at[p], sem)
        pltpu.async_copy(w_out.at[pi.at[pl.ds(c * CB, CB)]], vout2.at[p], sem)

        def gi(i, _):
            pltpu.async_copy(w_out.at[ni.at[c * CB + i]], vneg2.at[p, i], sem)
            return 0

        lax.fori_loop(0, CB, gi, 0)

    def drain(c, p):
        sem = sems.at[p]
        pltpu.make_async_copy(w_in.at[ci.at[pl.ds(c * CB, CB)]], vin2.at[p], sem).wait()
        pltpu.make_async_copy(w_out.at[pi.at[pl.ds(c * CB, CB)]], vout2.at[p], sem).wait()

        def gw(i, _):
            pltpu.make_async_copy(w_out.at[ni.at[c * CB + i]], vneg2.at[p, i], sem).wait()
            return 0

        lax.fori_loop(0, CB, gw, 0)

    def compute(c, p):
        lanes = lax.iota(jnp.int32, 16)

        def pos_group(g, _):
            for l in range(16):
                r = g * 16 + l
                plsc.store_scatter(tr, [lanes * 16 + l],
                                   _partial64(vin2.at[p], r, vout2.at[p], r))
            po[pl.ds(c * CB + g * 16, 16)] = _rowsum16(tr)
            return 0

        lax.fori_loop(0, CB // 16, pos_group, 0)

        def neg_group(g, _):
            for l in range(16):
                r = g * 16 + l
                i = r // K
                k = r - i * K
                a = vin2.at[p]
                b = vneg2.at[p]
                acc = a[i, pl.ds(0, 16)] * b[i, k, pl.ds(0, 16)]
                for j in range(1, 4):
                    acc = acc + a[i, pl.ds(j * 16, 16)] * b[i, k, pl.ds(j * 16, 16)]
                plsc.store_scatter(tr, [lanes * 16 + l], acc)
            flat = c * CB * K + g * 16 + lanes
            rows = flat // K
            plsc.store_scatter(no, [rows, flat - rows * K], _rowsum16(tr))
            return 0

        lax.fori_loop(0, CB * K // 16, neg_group, 0)

    issue(0, 0)

    def chunk_body(c, _):
        p = lax.rem(c, 2)

        @pl.when(c + 1 < NCHUNK)
        def _():
            issue(c + 1, lax.rem(c + 1, 2))

        drain(c, p)
        compute(c, p)
        return 0

    lax.fori_loop(0, NCHUNK, chunk_body, 0)
    pltpu.sync_copy(po, pos_out.at[pl.ds(wbase, BPW)])
    pltpu.sync_copy(no, neg_out.at[pl.ds(wbase, BPW)])


_sg_call = functools.partial(
    pl.kernel,
    out_type=[
        jax.ShapeDtypeStruct((B,), jnp.float32),
        jax.ShapeDtypeStruct((B, K), jnp.float32),
    ],
    mesh=plsc.VectorSubcoreMesh(core_axis_name="c", subcore_axis_name="s"),
    compiler_params=pltpu.CompilerParams(
        needs_layout_passes=False, use_tc_tiling_on_sc=False
    ),
    scratch_types=[
        pltpu.VMEM((BPW,), jnp.int32),
        pltpu.VMEM((BPW,), jnp.int32),
        pltpu.VMEM((BPW, K), jnp.int32),
        pltpu.VMEM((2, CB, DIM), jnp.float32),
        pltpu.VMEM((2, CB, DIM), jnp.float32),
        pltpu.VMEM((2, CB, K, DIM), jnp.float32),
        pltpu.VMEM((BPW,), jnp.float32),
        pltpu.VMEM((BPW, K), jnp.float32),
        pltpu.VMEM((256,), jnp.float32),
        pltpu.SemaphoreType.DMA((2,)),
    ],
)(_sg_body)


def kernel(center_words, pos_context_words, neg_context_words, W_in, W_out):
    cw = center_words.astype(jnp.int32)
    pw = pos_context_words.astype(jnp.int32)
    nw = neg_context_words.astype(jnp.int32)
    pos_scores, neg_scores = _sg_call(cw, pw, nw, W_in, W_out)
    return pos_scores, neg_scores
